# baseline (speedup 1.0000x reference)
import functools

import jax
import jax.numpy as jnp
from jax import lax
from jax.experimental import pallas as pl
from jax.experimental.pallas import tpu as pltpu
from jax.experimental.pallas import tpu_sc as plsc

B, H, W = 2, 128, 128
HW = H * W
L = (H // 2) * (W // 2)
C1, C2 = 384, 192
_BF = jnp.bfloat16
_F32 = jnp.float32

_T1 = 4096


def _conv_body(sfm_ref, x_ref, w1_ref, w2_ref, b1_ref, b2_ref, rec_ref, ref_ref):
    acc1 = None
    for c in range(C1 // 128):
        sb = sfm_ref[0, pl.ds(c * 128, 128), :].astype(_BF)
        wb = w1_ref[:, pl.ds(c * 128, 128)].astype(_BF)
        p = jax.lax.dot_general(wb, sb, (((1,), (0,)), ((), ())),
                                preferred_element_type=_F32)
        acc1 = p if acc1 is None else acc1 + p
    rec_ref[0] = jnp.maximum(acc1 + b1_ref[0, 0], 0.0)

    xb = x_ref[0].astype(_BF)
    w2b = w2_ref[...].astype(_BF)
    acc2 = jax.lax.dot_general(w2b, xb, (((1,), (0,)), ((), ())),
                               preferred_element_type=_F32)
    ref_ref[0] = jnp.maximum(acc2 + b2_ref[0, 0], 0.0)


def _convs(sfm, x, w1r, w2r, b1, b2):
    grid = (B, HW // _T1)
    return pl.pallas_call(
        _conv_body,
        grid=grid,
        in_specs=[
            pl.BlockSpec((1, C1, _T1), lambda ib, it: (ib, 0, it)),
            pl.BlockSpec((1, C2, _T1), lambda ib, it: (ib, 0, it)),
            pl.BlockSpec((1, C1), lambda ib, it: (0, 0)),
            pl.BlockSpec((1, C2), lambda ib, it: (0, 0)),
            pl.BlockSpec((1, 1), lambda ib, it: (0, 0), memory_space=pltpu.SMEM),
            pl.BlockSpec((1, 1), lambda ib, it: (0, 0), memory_space=pltpu.SMEM),
        ],
        out_specs=[
            pl.BlockSpec((1, 1, _T1), lambda ib, it: (ib, 0, it)),
            pl.BlockSpec((1, 1, _T1), lambda ib, it: (ib, 0, it)),
        ],
        out_shape=[
            jax.ShapeDtypeStruct((B, 1, HW), _F32),
            jax.ShapeDtypeStruct((B, 1, HW), _F32),
        ],
    )(sfm, x, w1r, w2r, b1, b2)


_LT = 128
_PT = 4096
_NG = HW // 4


def _corr_body(pb_ref, g_ref, idx_ref):
    pb = pb_ref[0]

    def body(c, carry):
        rm, ra = carry
        base = c * _PT
        gm = None
        for ph in range(4):
            gph = g_ref[0, pl.ds(4 * ph, 4), pl.ds(base, _PT)]
            y = jax.lax.dot_general(pb, gph, (((1,), (0,)), ((), ())),
                                    preferred_element_type=_F32)
            gm = y if gm is None else jnp.maximum(gm, y)
        m = jnp.max(gm, axis=1, keepdims=True)
        io = jax.lax.broadcasted_iota(jnp.int32, (_LT, _PT), 1) + base
        cand = jnp.min(jnp.where(gm == m, io, _NG), axis=1, keepdims=True)
        upd = m > rm
        return (jnp.where(upd, m, rm), jnp.where(upd, cand, ra))

    rm0 = jnp.full((_LT, 1), -jnp.inf, _F32)
    ra0 = jnp.zeros((_LT, 1), jnp.int32)
    _, ra = jax.lax.fori_loop(0, _NG // _PT, body, (rm0, ra0))
    idx_ref[0] = ra


def _corr(pb, g):
    grid = (B, L // _LT)
    return pl.pallas_call(
        _corr_body,
        grid=grid,
        in_specs=[
            pl.BlockSpec((1, _LT, 4), lambda ib, il: (ib, il, 0)),
            pl.BlockSpec((1, 16, _NG), lambda ib, il: (ib, 0, 0)),
        ],
        out_specs=pl.BlockSpec((1, _LT, 1), lambda ib, il: (ib, il, 0)),
        out_shape=jax.ShapeDtypeStruct((B, L, 1), jnp.int32),
    )(pb, g)


_DT = 256
_ST = 512


def _gather_body(idx_ref, pg_ref, out_ref):
    idx = idx_ref[0]

    def body(s, acc):
        base = s * _ST
        io = jax.lax.broadcasted_iota(jnp.int32, (_DT, _ST), 1) + base
        oh = io == idx
        cols = []
        for k in range(4):
            pgk = pg_ref[0, k:k + 1, pl.ds(base, _ST)]
            cols.append(jnp.sum(jnp.where(oh, pgk, 0.0), axis=1, keepdims=True))
        return acc + jnp.concatenate(cols, axis=1)

    acc0 = jnp.zeros((_DT, 4), _F32)
    out_ref[0] = jax.lax.fori_loop(0, L // _ST, body, acc0)


def _gather(idx, pg):
    grid = (B, L // _DT)
    return pl.pallas_call(
        _gather_body,
        grid=grid,
        in_specs=[
            pl.BlockSpec((1, _DT, 1), lambda ib, il: (ib, il, 0)),
            pl.BlockSpec((1, 4, L), lambda ib, il: (ib, 0, 0)),
        ],
        out_specs=pl.BlockSpec((1, _DT, 4), lambda ib, il: (ib, il, 0)),
        out_shape=jax.ShapeDtypeStruct((B, L, 4), _F32),
    )(idx, pg)


_NROWS = B * L
_D = 128


def _sc_gather(idx_flat, table):
    info = plsc.get_sparse_core_info()
    nw = info.num_cores * info.num_subcores
    bpw = _NROWS // nw
    mesh = plsc.VectorSubcoreMesh(core_axis_name="c", subcore_axis_name="s")

    @functools.partial(
        pl.kernel, mesh=mesh,
        out_type=jax.ShapeDtypeStruct((_NROWS, _D), _F32),
        scratch_types=[
            pltpu.VMEM((bpw,), jnp.int32),
            pltpu.VMEM((bpw, _D), _F32),
            pltpu.SemaphoreType.DMA,
        ],
    )
    def k(idx_hbm, table_hbm, out_hbm, idx_v, rows_v, sem):
        wid = lax.axis_index("s") * info.num_cores + lax.axis_index("c")
        base = wid * bpw
        pltpu.sync_copy(idx_hbm.at[pl.ds(base, bpw)], idx_v)
        pltpu.async_copy(table_hbm.at[idx_v], rows_v, sem).wait()
        pltpu.sync_copy(rows_v, out_hbm.at[pl.ds(base, bpw)])

    return k(idx_flat, table)


def kernel(spade_fm, x, w1, b1, w2, b2):
    sfm = spade_fm.reshape(B, C1, HW)
    xf = x.reshape(B, C2, HW)
    w1r = w1.reshape(1, C1)
    w2r = w2.reshape(1, C2)
    b1s = b1.reshape(1, 1)
    b2s = b2.reshape(1, 1)

    rec_flat, ref_flat = _convs(sfm, xf, w1r, w2r, b1s, b2s)
    rec_img = rec_flat.reshape(B, H, W)
    ref_img = ref_flat.reshape(B, H, W)

    pb = (ref_img.reshape(B, 64, 2, 64, 2)
          .transpose(0, 1, 3, 2, 4)
          .reshape(B, L, 4)).astype(_BF)
    gg = jnp.pad(rec_img, ((0, 0), (0, 1), (0, 1)))
    g = jnp.stack([gg[:, :H, :W], gg[:, :H, 1:W + 1],
                   gg[:, 1:H + 1, :W], gg[:, 1:H + 1, 1:W + 1]],
                  axis=1)
    g = (g.reshape(B, 4, H, W // 4, 4)
         .transpose(0, 4, 1, 2, 3)
         .reshape(B, 16, _NG).astype(_BF))

    idx = _corr(pb, g)

    pg = (rec_img.reshape(B, 64, 2, 64, 2)
          .transpose(0, 1, 3, 2, 4)
          .reshape(B, L, 4))
    table = jnp.pad(pg, ((0, 0), (0, 0), (0, _D - 4))).reshape(_NROWS, _D)
    idx_flat = (idx[:, :, 0] + (jnp.arange(B, dtype=jnp.int32) * L)[:, None]
                ).reshape(_NROWS)

    out4 = _sc_gather(idx_flat, table)[:, :4].reshape(B, L, 4)

    out = (out4.reshape(B, 64, 64, 2, 2)
           .transpose(0, 1, 3, 2, 4)
           .reshape(B, 1, H, W))
    return out


# speedup vs baseline: 1.0464x; 1.0464x over previous
import functools

import jax
import jax.numpy as jnp
from jax import lax
from jax.experimental import pallas as pl
from jax.experimental.pallas import tpu as pltpu
from jax.experimental.pallas import tpu_sc as plsc

B, H, W = 2, 128, 128
HW = H * W
L = (H // 2) * (W // 2)
C1, C2 = 384, 192
_BF = jnp.bfloat16
_F32 = jnp.float32

_T1 = 4096


def _conv_body(sfm_ref, x_ref, w1_ref, w2_ref, b1_ref, b2_ref, rec_ref, ref_ref):
    acc1 = None
    for c in range(C1 // 128):
        sb = sfm_ref[0, pl.ds(c * 128, 128), :].astype(_BF)
        wb = w1_ref[:, pl.ds(c * 128, 128)].astype(_BF)
        p = jax.lax.dot_general(wb, sb, (((1,), (0,)), ((), ())),
                                preferred_element_type=_F32)
        acc1 = p if acc1 is None else acc1 + p
    rec_ref[0] = jnp.maximum(acc1 + b1_ref[0, 0], 0.0)

    xb = x_ref[0].astype(_BF)
    w2b = w2_ref[...].astype(_BF)
    acc2 = jax.lax.dot_general(w2b, xb, (((1,), (0,)), ((), ())),
                               preferred_element_type=_F32)
    ref_ref[0] = jnp.maximum(acc2 + b2_ref[0, 0], 0.0)


def _convs(sfm, x, w1r, w2r, b1, b2):
    grid = (B, HW // _T1)
    return pl.pallas_call(
        _conv_body,
        grid=grid,
        in_specs=[
            pl.BlockSpec((1, C1, _T1), lambda ib, it: (ib, 0, it)),
            pl.BlockSpec((1, C2, _T1), lambda ib, it: (ib, 0, it)),
            pl.BlockSpec((1, C1), lambda ib, it: (0, 0)),
            pl.BlockSpec((1, C2), lambda ib, it: (0, 0)),
            pl.BlockSpec((1, 1), lambda ib, it: (0, 0), memory_space=pltpu.SMEM),
            pl.BlockSpec((1, 1), lambda ib, it: (0, 0), memory_space=pltpu.SMEM),
        ],
        out_specs=[
            pl.BlockSpec((1, 1, _T1), lambda ib, it: (ib, 0, it)),
            pl.BlockSpec((1, 1, _T1), lambda ib, it: (ib, 0, it)),
        ],
        out_shape=[
            jax.ShapeDtypeStruct((B, 1, HW), _F32),
            jax.ShapeDtypeStruct((B, 1, HW), _F32),
        ],
    )(sfm, x, w1r, w2r, b1, b2)


_LT = 512
_PT = 4096
_NG = HW // 4


def _corr_body(pb_ref, g_ref, idx_ref):
    pb = pb_ref[0]

    def body(c, carry):
        rm, ra = carry
        base = c * _PT
        gm = None
        for ph in range(4):
            gph = g_ref[0, pl.ds(4 * ph, 4), pl.ds(base, _PT)]
            y = jax.lax.dot_general(pb, gph, (((1,), (0,)), ((), ())),
                                    preferred_element_type=_F32)
            gm = y if gm is None else jnp.maximum(gm, y)
        m = jnp.max(gm, axis=1, keepdims=True)
        io = jax.lax.broadcasted_iota(jnp.int32, (_LT, _PT), 1) + base
        cand = jnp.min(jnp.where(gm == m, io, _NG), axis=1, keepdims=True)
        upd = m > rm
        return (jnp.where(upd, m, rm), jnp.where(upd, cand, ra))

    rm0 = jnp.full((_LT, 1), -jnp.inf, _F32)
    ra0 = jnp.zeros((_LT, 1), jnp.int32)
    _, ra = jax.lax.fori_loop(0, _NG // _PT, body, (rm0, ra0))
    idx_ref[0] = ra


def _corr(pb, g):
    grid = (B, L // _LT)
    return pl.pallas_call(
        _corr_body,
        grid=grid,
        in_specs=[
            pl.BlockSpec((1, _LT, 4), lambda ib, il: (ib, il, 0)),
            pl.BlockSpec((1, 16, _NG), lambda ib, il: (ib, 0, 0)),
        ],
        out_specs=pl.BlockSpec((1, _LT, 1), lambda ib, il: (ib, il, 0)),
        out_shape=jax.ShapeDtypeStruct((B, L, 1), jnp.int32),
    )(pb, g)


_DT = 256
_ST = 512


def _gather_body(idx_ref, pg_ref, out_ref):
    idx = idx_ref[0]

    def body(s, acc):
        base = s * _ST
        io = jax.lax.broadcasted_iota(jnp.int32, (_DT, _ST), 1) + base
        oh = io == idx
        cols = []
        for k in range(4):
            pgk = pg_ref[0, k:k + 1, pl.ds(base, _ST)]
            cols.append(jnp.sum(jnp.where(oh, pgk, 0.0), axis=1, keepdims=True))
        return acc + jnp.concatenate(cols, axis=1)

    acc0 = jnp.zeros((_DT, 4), _F32)
    out_ref[0] = jax.lax.fori_loop(0, L // _ST, body, acc0)


def _gather(idx, pg):
    grid = (B, L // _DT)
    return pl.pallas_call(
        _gather_body,
        grid=grid,
        in_specs=[
            pl.BlockSpec((1, _DT, 1), lambda ib, il: (ib, il, 0)),
            pl.BlockSpec((1, 4, L), lambda ib, il: (ib, 0, 0)),
        ],
        out_specs=pl.BlockSpec((1, _DT, 4), lambda ib, il: (ib, il, 0)),
        out_shape=jax.ShapeDtypeStruct((B, L, 4), _F32),
    )(idx, pg)


_NROWS = B * L
_D = 128


def _sc_gather(idx_flat, table):
    info = plsc.get_sparse_core_info()
    nw = info.num_cores * info.num_subcores
    bpw = _NROWS // nw
    mesh = plsc.VectorSubcoreMesh(core_axis_name="c", subcore_axis_name="s")

    @functools.partial(
        pl.kernel, mesh=mesh,
        out_type=jax.ShapeDtypeStruct((_NROWS, _D), _F32),
        scratch_types=[
            pltpu.VMEM((bpw,), jnp.int32),
            pltpu.VMEM((bpw, _D), _F32),
            pltpu.SemaphoreType.DMA,
        ],
    )
    def k(idx_hbm, table_hbm, out_hbm, idx_v, rows_v, sem):
        wid = lax.axis_index("s") * info.num_cores + lax.axis_index("c")
        base = wid * bpw
        pltpu.sync_copy(idx_hbm.at[pl.ds(base, bpw)], idx_v)
        pltpu.async_copy(table_hbm.at[idx_v], rows_v, sem).wait()
        pltpu.sync_copy(rows_v, out_hbm.at[pl.ds(base, bpw)])

    return k(idx_flat, table)


def kernel(spade_fm, x, w1, b1, w2, b2):
    sfm = spade_fm.reshape(B, C1, HW)
    xf = x.reshape(B, C2, HW)
    w1r = w1.reshape(1, C1)
    w2r = w2.reshape(1, C2)
    b1s = b1.reshape(1, 1)
    b2s = b2.reshape(1, 1)

    rec_flat, ref_flat = _convs(sfm, xf, w1r, w2r, b1s, b2s)
    rec_img = rec_flat.reshape(B, H, W)
    ref_img = ref_flat.reshape(B, H, W)

    pb = (ref_img.reshape(B, 64, 2, 64, 2)
          .transpose(0, 1, 3, 2, 4)
          .reshape(B, L, 4)).astype(_BF)
    gg = jnp.pad(rec_img, ((0, 0), (0, 1), (0, 1)))
    g = jnp.stack([gg[:, :H, :W], gg[:, :H, 1:W + 1],
                   gg[:, 1:H + 1, :W], gg[:, 1:H + 1, 1:W + 1]],
                  axis=1)
    g = (g.reshape(B, 4, H, W // 4, 4)
         .transpose(0, 4, 1, 2, 3)
         .reshape(B, 16, _NG).astype(_BF))

    idx = _corr(pb, g)

    pg = (rec_img.reshape(B, 64, 2, 64, 2)
          .transpose(0, 1, 3, 2, 4)
          .reshape(B, L, 4))
    table = jnp.pad(pg, ((0, 0), (0, 0), (0, _D - 4))).reshape(_NROWS, _D)
    idx_flat = (idx[:, :, 0] + (jnp.arange(B, dtype=jnp.int32) * L)[:, None]
                ).reshape(_NROWS)

    out4 = _sc_gather(idx_flat, table)[:, :4].reshape(B, L, 4)

    out = (out4.reshape(B, 64, 64, 2, 2)
           .transpose(0, 1, 3, 2, 4)
           .reshape(B, 1, H, W))
    return out


# speedup vs baseline: 1.0538x; 1.0071x over previous
import functools

import jax
import jax.numpy as jnp
from jax import lax
from jax.experimental import pallas as pl
from jax.experimental.pallas import tpu as pltpu
from jax.experimental.pallas import tpu_sc as plsc

B, H, W = 2, 128, 128
HW = H * W
L = (H // 2) * (W // 2)
C1, C2 = 384, 192
_BF = jnp.bfloat16
_F32 = jnp.float32

_T1 = 4096


def _conv_body(sfm_ref, x_ref, w1_ref, w2_ref, b1_ref, b2_ref, rec_ref, ref_ref):
    acc1 = None
    for c in range(C1 // 128):
        sb = sfm_ref[0, pl.ds(c * 128, 128), :].astype(_BF)
        wb = w1_ref[:, pl.ds(c * 128, 128)].astype(_BF)
        p = jax.lax.dot_general(wb, sb, (((1,), (0,)), ((), ())),
                                preferred_element_type=_F32)
        acc1 = p if acc1 is None else acc1 + p
    rec_ref[0] = jnp.maximum(acc1 + b1_ref[0, 0], 0.0)

    xb = x_ref[0].astype(_BF)
    w2b = w2_ref[...].astype(_BF)
    acc2 = jax.lax.dot_general(w2b, xb, (((1,), (0,)), ((), ())),
                               preferred_element_type=_F32)
    ref_ref[0] = jnp.maximum(acc2 + b2_ref[0, 0], 0.0)


def _convs(sfm, x, w1r, w2r, b1, b2):
    grid = (B, HW // _T1)
    return pl.pallas_call(
        _conv_body,
        grid=grid,
        in_specs=[
            pl.BlockSpec((1, C1, _T1), lambda ib, it: (ib, 0, it)),
            pl.BlockSpec((1, C2, _T1), lambda ib, it: (ib, 0, it)),
            pl.BlockSpec((1, C1), lambda ib, it: (0, 0)),
            pl.BlockSpec((1, C2), lambda ib, it: (0, 0)),
            pl.BlockSpec((1, 1), lambda ib, it: (0, 0), memory_space=pltpu.SMEM),
            pl.BlockSpec((1, 1), lambda ib, it: (0, 0), memory_space=pltpu.SMEM),
        ],
        out_specs=[
            pl.BlockSpec((1, 1, _T1), lambda ib, it: (ib, 0, it)),
            pl.BlockSpec((1, 1, _T1), lambda ib, it: (ib, 0, it)),
        ],
        out_shape=[
            jax.ShapeDtypeStruct((B, 1, HW), _F32),
            jax.ShapeDtypeStruct((B, 1, HW), _F32),
        ],
    )(sfm, x, w1r, w2r, b1, b2)


_LT = 1024
_PT = 4096
_NG = HW // 4


def _corr_body(pb_ref, g_ref, idx_ref):
    pb = pb_ref[0]

    def body(c, carry):
        rm, ra = carry
        base = c * _PT
        gm = None
        for ph in range(4):
            gph = g_ref[0, pl.ds(4 * ph, 4), pl.ds(base, _PT)]
            y = jax.lax.dot_general(pb, gph, (((1,), (0,)), ((), ())),
                                    preferred_element_type=_F32)
            gm = y if gm is None else jnp.maximum(gm, y)
        m = jnp.max(gm, axis=1, keepdims=True)
        io = jax.lax.broadcasted_iota(jnp.int32, (_LT, _PT), 1) + base
        cand = jnp.min(jnp.where(gm == m, io, _NG), axis=1, keepdims=True)
        upd = m > rm
        return (jnp.where(upd, m, rm), jnp.where(upd, cand, ra))

    rm0 = jnp.full((_LT, 1), -jnp.inf, _F32)
    ra0 = jnp.zeros((_LT, 1), jnp.int32)
    _, ra = jax.lax.fori_loop(0, _NG // _PT, body, (rm0, ra0))
    idx_ref[0] = ra


def _corr(pb, g):
    grid = (B, L // _LT)
    return pl.pallas_call(
        _corr_body,
        grid=grid,
        in_specs=[
            pl.BlockSpec((1, _LT, 4), lambda ib, il: (ib, il, 0)),
            pl.BlockSpec((1, 16, _NG), lambda ib, il: (ib, 0, 0)),
        ],
        out_specs=pl.BlockSpec((1, _LT, 1), lambda ib, il: (ib, il, 0)),
        out_shape=jax.ShapeDtypeStruct((B, L, 1), jnp.int32),
    )(pb, g)


_DT = 256
_ST = 512


def _gather_body(idx_ref, pg_ref, out_ref):
    idx = idx_ref[0]

    def body(s, acc):
        base = s * _ST
        io = jax.lax.broadcasted_iota(jnp.int32, (_DT, _ST), 1) + base
        oh = io == idx
        cols = []
        for k in range(4):
            pgk = pg_ref[0, k:k + 1, pl.ds(base, _ST)]
            cols.append(jnp.sum(jnp.where(oh, pgk, 0.0), axis=1, keepdims=True))
        return acc + jnp.concatenate(cols, axis=1)

    acc0 = jnp.zeros((_DT, 4), _F32)
    out_ref[0] = jax.lax.fori_loop(0, L // _ST, body, acc0)


def _gather(idx, pg):
    grid = (B, L // _DT)
    return pl.pallas_call(
        _gather_body,
        grid=grid,
        in_specs=[
            pl.BlockSpec((1, _DT, 1), lambda ib, il: (ib, il, 0)),
            pl.BlockSpec((1, 4, L), lambda ib, il: (ib, 0, 0)),
        ],
        out_specs=pl.BlockSpec((1, _DT, 4), lambda ib, il: (ib, il, 0)),
        out_shape=jax.ShapeDtypeStruct((B, L, 4), _F32),
    )(idx, pg)


_NROWS = B * L
_D = 128


def _sc_gather(idx_flat, table):
    info = plsc.get_sparse_core_info()
    nw = info.num_cores * info.num_subcores
    bpw = _NROWS // nw
    mesh = plsc.VectorSubcoreMesh(core_axis_name="c", subcore_axis_name="s")

    @functools.partial(
        pl.kernel, mesh=mesh,
        out_type=jax.ShapeDtypeStruct((_NROWS, _D), _F32),
        scratch_types=[
            pltpu.VMEM((bpw,), jnp.int32),
            pltpu.VMEM((bpw, _D), _F32),
            pltpu.SemaphoreType.DMA,
        ],
    )
    def k(idx_hbm, table_hbm, out_hbm, idx_v, rows_v, sem):
        wid = lax.axis_index("s") * info.num_cores + lax.axis_index("c")
        base = wid * bpw
        pltpu.sync_copy(idx_hbm.at[pl.ds(base, bpw)], idx_v)
        pltpu.async_copy(table_hbm.at[idx_v], rows_v, sem).wait()
        pltpu.sync_copy(rows_v, out_hbm.at[pl.ds(base, bpw)])

    return k(idx_flat, table)


def kernel(spade_fm, x, w1, b1, w2, b2):
    sfm = spade_fm.reshape(B, C1, HW)
    xf = x.reshape(B, C2, HW)
    w1r = w1.reshape(1, C1)
    w2r = w2.reshape(1, C2)
    b1s = b1.reshape(1, 1)
    b2s = b2.reshape(1, 1)

    rec_flat, ref_flat = _convs(sfm, xf, w1r, w2r, b1s, b2s)
    rec_img = rec_flat.reshape(B, H, W)
    ref_img = ref_flat.reshape(B, H, W)

    pb = (ref_img.reshape(B, 64, 2, 64, 2)
          .transpose(0, 1, 3, 2, 4)
          .reshape(B, L, 4)).astype(_BF)
    gg = jnp.pad(rec_img, ((0, 0), (0, 1), (0, 1)))
    g = jnp.stack([gg[:, :H, :W], gg[:, :H, 1:W + 1],
                   gg[:, 1:H + 1, :W], gg[:, 1:H + 1, 1:W + 1]],
                  axis=1)
    g = (g.reshape(B, 4, H, W // 4, 4)
         .transpose(0, 4, 1, 2, 3)
         .reshape(B, 16, _NG).astype(_BF))

    idx = _corr(pb, g)

    pg = (rec_img.reshape(B, 64, 2, 64, 2)
          .transpose(0, 1, 3, 2, 4)
          .reshape(B, L, 4))
    table = jnp.pad(pg, ((0, 0), (0, 0), (0, _D - 4))).reshape(_NROWS, _D)
    idx_flat = (idx[:, :, 0] + (jnp.arange(B, dtype=jnp.int32) * L)[:, None]
                ).reshape(_NROWS)

    out4 = _sc_gather(idx_flat, table)[:, :4].reshape(B, L, 4)

    out = (out4.reshape(B, 64, 64, 2, 2)
           .transpose(0, 1, 3, 2, 4)
           .reshape(B, 1, H, W))
    return out
